# Initial kernel scaffold; baseline (speedup 1.0000x reference)
import jax
import jax.numpy as jnp
from jax.experimental import pallas as pl


def kernel(uid, seq, pos, neg, nbr, nbr_iid, user_embs, item_embs):
    raise NotImplementedError("write your pallas kernel here")


# speedup vs baseline: 5.0629x; 5.0629x over previous
import functools

import jax
import jax.numpy as jnp
from jax import lax
from jax.experimental import pallas as pl
from jax.experimental.pallas import tpu as pltpu
from jax.experimental.pallas import tpu_sc as plsc

B = 4096
L = 50
D = 64
NL = 16
NC = D // NL
NW = 32
RPW = B // NW
GCH = 2
GR = GCH * L
SC_ROWS = 4
NSC = RPW // SC_ROWS
SR = SC_ROWS * L
NG = RPW // GCH
LP = 64

_mesh = plsc.VectorSubcoreMesh(core_axis_name="c", subcore_axis_name="s")

_f32 = jnp.float32
_out_row = jax.ShapeDtypeStruct((B * L, D), _f32)


@functools.partial(
    pl.kernel,
    mesh=_mesh,
    compiler_params=pltpu.CompilerParams(
        needs_layout_passes=False, use_tc_tiling_on_sc=False),
    out_type=[
        _out_row,
        _out_row,
        _out_row,
        _out_row,
        jax.ShapeDtypeStruct((NW, NSC, SC_ROWS * LP), _f32),
        jax.ShapeDtypeStruct((NW, NSC, SC_ROWS * LP), _f32),
    ],
    scratch_types=[
        pltpu.VMEM((RPW,), jnp.int32),
        pltpu.VMEM((RPW, D), _f32),
        pltpu.VMEM((8, D), _f32),
        pltpu.VMEM((NG, GR), jnp.int32),
        pltpu.VMEM((NG, GR), jnp.int32),
        pltpu.VMEM((NG, GR), jnp.int32),
        pltpu.VMEM((SR, D), _f32),
        pltpu.VMEM((SR, D), _f32),
        pltpu.VMEM((SR, D), _f32),
        pltpu.VMEM((SR, D), _f32),
        pltpu.VMEM((SR, D), _f32),
        pltpu.VMEM((NSC, SC_ROWS * LP), _f32),
        pltpu.VMEM((NSC, SC_ROWS * LP), _f32),
        pltpu.SemaphoreType.DMA,
        pltpu.SemaphoreType.DMA,
        pltpu.SemaphoreType.DMA,
        pltpu.SemaphoreType.DMA,
        pltpu.SemaphoreType.DMA,
    ],
)
def _social_mf_sc(uid_f, pos_r, neg_r, nbr_r, user_e, item_e,
                  hu_o, pos_o, neg_o, nbr_o, plog_o, nlog_o,
                  uidx_v, u_rows, u0_v, pidx_v, nidx_v, bidx_v,
                  pos_v, neg_v, nbr_v, hu_b, nbr_b,
                  plog_v, nlog_v,
                  sem_u, sem_p, sem_n, sem_b, sem_w):
    wid = lax.axis_index("s") * 2 + lax.axis_index("c")
    iota = lax.iota(jnp.int32, NL)

    pltpu.sync_copy(uid_f.at[pl.ds(wid * RPW, RPW)], uidx_v)
    pltpu.sync_copy(pos_r.at[wid], pidx_v)
    pltpu.sync_copy(neg_r.at[wid], nidx_v)
    pltpu.sync_copy(nbr_r.at[wid], bidx_v)
    pltpu.sync_copy(user_e.at[pl.ds(0, 8)], u0_v)
    pltpu.async_copy(user_e.at[uidx_v], u_rows, sem_u).wait()
    u0 = [u0_v[0, pl.ds(c * NL, NL)] for c in range(NC)]

    def chunk_body(q, carry):
        copies = []
        for h in range(SC_ROWS // GCH):
            gi = q * (SC_ROWS // GCH) + h
            dst = pl.ds(h * GR, GR)
            copies.append(pltpu.async_copy(
                item_e.at[pidx_v.at[gi]], pos_v.at[dst], sem_p))
            copies.append(pltpu.async_copy(
                item_e.at[nidx_v.at[gi]], neg_v.at[dst], sem_n))
            copies.append(pltpu.async_copy(
                user_e.at[bidx_v.at[gi]], nbr_v.at[dst], sem_b))
        for c in copies:
            c.wait()

        for r in range(SC_ROWS):
            row = q * SC_ROWS + r
            u = [u_rows[row, pl.ds(c * NL, NL)] for c in range(NC)]
            gi = q * (SC_ROWS // GCH) + r // GCH
            goff = (r % GCH) * L

            def l_body(l, acc, r=r, u=u):
                a0, a1, a2, a3, plv, nlv = acc
                g = r * L + l
                pv = [pos_v[g, pl.ds(c * NL, NL)] for c in range(NC)]
                nv = [neg_v[g, pl.ds(c * NL, NL)] for c in range(NC)]
                bv = [nbr_v[g, pl.ds(c * NL, NL)] for c in range(NC)]
                ps = jnp.sum(u[0] * pv[0] + u[1] * pv[1]
                             + u[2] * pv[2] + u[3] * pv[3])
                ns = jnp.sum(u[0] * nv[0] + u[1] * nv[1]
                             + u[2] * nv[2] + u[3] * nv[3])
                eq = iota == (l % NL)
                plv = jnp.where(eq, ps, plv)
                nlv = jnp.where(eq, ns, nlv)
                slot = r * LP + (l // NL) * NL
                plog_v[q, pl.ds(slot, NL)] = plv
                nlog_v[q, pl.ds(slot, NL)] = nlv
                for c in range(NC):
                    hu_b[g, pl.ds(c * NL, NL)] = u[c]
                a0 = a0 + bv[0]
                a1 = a1 + bv[1]
                a2 = a2 + bv[2]
                a3 = a3 + bv[3]
                return (a0, a1, a2, a3, plv, nlv)

            z = jnp.zeros((NL,), _f32)
            a0, a1, a2, a3, _, _ = lax.fori_loop(
                0, L, l_body, (z, z, z, z, z, z))

            zc = jnp.zeros((NL,), jnp.int32)
            for k in range(3):
                bvix = bidx_v[gi, pl.ds(goff + k * NL, NL)]
                zc = zc + jnp.where(bvix == 0, jnp.int32(1), jnp.int32(0))
            tail = bidx_v[gi, pl.ds(goff + 34, NL)]
            tmask = (tail == 0) & (iota >= NL - 2)
            zc = zc + jnp.where(tmask, jnp.int32(1), jnp.int32(0))
            nzero = jnp.sum(zc)
            nzf = nzero.astype(_f32)
            cf = _f32(L) - nzf
            nonempty = nzero < L
            a = [a0, a1, a2, a3]
            m = [jnp.where(nonempty, (a[c] - nzf * u0[c]) / cf,
                           jnp.zeros((NL,), _f32) / cf)
                 for c in range(NC)]

            def fill_body(l, _, r=r, m=m):
                g = r * L + l
                for c in range(NC):
                    nbr_b[g, pl.ds(c * NL, NL)] = m[c]
                return 0

            lax.fori_loop(0, L, fill_body, 0)

        start = wid * (RPW * L) + q * SR
        w1 = pltpu.async_copy(pos_v, pos_o.at[pl.ds(start, SR)], sem_w)
        w2 = pltpu.async_copy(neg_v, neg_o.at[pl.ds(start, SR)], sem_w)
        w3 = pltpu.async_copy(hu_b, hu_o.at[pl.ds(start, SR)], sem_w)
        w4 = pltpu.async_copy(nbr_b, nbr_o.at[pl.ds(start, SR)], sem_w)
        w1.wait()
        w2.wait()
        w3.wait()
        w4.wait()
        return carry

    lax.fori_loop(0, NSC, chunk_body, 0)
    pltpu.sync_copy(plog_v, plog_o.at[wid])
    pltpu.sync_copy(nlog_v, nlog_o.at[wid])


def kernel(uid, seq, pos, neg, nbr, nbr_iid, user_embs, item_embs):
    del seq, nbr_iid
    uid_f = uid.astype(jnp.int32)
    pos_r = pos.astype(jnp.int32).reshape(NW, NG, GR)
    neg_r = neg.astype(jnp.int32).reshape(NW, NG, GR)
    nbr_r = nbr.astype(jnp.int32).reshape(NW, NG, GR)
    hu, pos_hi, neg_hi, nbr_emb, plog, nlog = _social_mf_sc(
        uid_f, pos_r, neg_r, nbr_r, user_embs, item_embs)
    return (
        plog.reshape(B, LP)[:, :L],
        nlog.reshape(B, LP)[:, :L],
        hu.reshape(B, L, D),
        pos_hi.reshape(B, L, D),
        neg_hi.reshape(B, L, D),
        nbr_emb.reshape(B, L, D),
    )


# speedup vs baseline: 5.9291x; 1.1711x over previous
import functools

import jax
import jax.numpy as jnp
from jax import lax
from jax.experimental import pallas as pl
from jax.experimental.pallas import tpu as pltpu
from jax.experimental.pallas import tpu_sc as plsc

B = 4096
L = 50
D = 64
NL = 16
NC = D // NL
NW = 32
RPW = B // NW
CH = 2
NSC = RPW // CH
SR = CH * L
LP = 64
NSET = 3

_mesh = plsc.VectorSubcoreMesh(core_axis_name="c", subcore_axis_name="s")

_f32 = jnp.float32
_out_row = jax.ShapeDtypeStruct((B * L, D), _f32)


def _ring_scratch():
    per_set = [
        pltpu.VMEM((CH, D), _f32),
        pltpu.VMEM((SR, D), _f32),
        pltpu.VMEM((SR, D), _f32),
        pltpu.VMEM((SR, D), _f32),
        pltpu.VMEM((SR, D), _f32),
        pltpu.VMEM((SR, D), _f32),
        pltpu.VMEM((CH * LP,), _f32),
        pltpu.VMEM((CH * LP,), _f32),
        pltpu.SemaphoreType.DMA,
        pltpu.SemaphoreType.DMA,
    ]
    return per_set * NSET


@functools.partial(
    pl.kernel,
    mesh=_mesh,
    compiler_params=pltpu.CompilerParams(
        needs_layout_passes=False, use_tc_tiling_on_sc=False),
    out_type=[
        _out_row,
        _out_row,
        _out_row,
        _out_row,
        jax.ShapeDtypeStruct((NW, NSC, CH * LP), _f32),
        jax.ShapeDtypeStruct((NW, NSC, CH * LP), _f32),
    ],
    scratch_types=[
        pltpu.VMEM((NSC, CH), jnp.int32),
        pltpu.VMEM((8, D), _f32),
        pltpu.VMEM((NSC, SR), jnp.int32),
        pltpu.VMEM((NSC, SR), jnp.int32),
        pltpu.VMEM((NSC, SR), jnp.int32),
        pltpu.SemaphoreType.DMA,
    ] + _ring_scratch(),
)
def _social_mf_sc(uid_r, pos_r, neg_r, nbr_r, user_e, item_e,
                  hu_o, pos_o, neg_o, nbr_o, plog_o, nlog_o,
                  uidx_v, u0_v, pidx_v, nidx_v, bidx_v, sem_misc,
                  *ring):
    wid = lax.axis_index("s") * 2 + lax.axis_index("c")
    iota = lax.iota(jnp.int32, NL)
    sets = [ring[i * 10:(i + 1) * 10] for i in range(NSET)]

    pltpu.sync_copy(uid_r.at[wid], uidx_v)
    pltpu.sync_copy(pos_r.at[wid], pidx_v)
    pltpu.sync_copy(neg_r.at[wid], nidx_v)
    pltpu.sync_copy(nbr_r.at[wid], bidx_v)
    pltpu.sync_copy(user_e.at[pl.ds(0, 8)], u0_v)
    u0 = [u0_v[0, pl.ds(c * NL, NL)] for c in range(NC)]

    def g_descs(q, s):
        u_b, pos_v, neg_v, nbr_v = sets[s][0], sets[s][1], sets[s][2], sets[s][3]
        sem_g = sets[s][8]
        return [
            pltpu.make_async_copy(user_e.at[uidx_v.at[q]], u_b, sem_g),
            pltpu.make_async_copy(item_e.at[pidx_v.at[q]], pos_v, sem_g),
            pltpu.make_async_copy(item_e.at[nidx_v.at[q]], neg_v, sem_g),
            pltpu.make_async_copy(user_e.at[bidx_v.at[q]], nbr_v, sem_g),
        ]

    def w_descs(q, s):
        (_, pos_v, neg_v, nbr_v, hu_b, nbr_b, plog_b, nlog_b, _, sem_w) = sets[s]
        sl = pl.ds(wid * (RPW * L) + q * SR, SR)
        return [
            pltpu.make_async_copy(pos_v, pos_o.at[sl], sem_w),
            pltpu.make_async_copy(neg_v, neg_o.at[sl], sem_w),
            pltpu.make_async_copy(hu_b, hu_o.at[sl], sem_w),
            pltpu.make_async_copy(nbr_b, nbr_o.at[sl], sem_w),
            pltpu.make_async_copy(plog_b, plog_o.at[wid, q], sem_w),
            pltpu.make_async_copy(nlog_b, nlog_o.at[wid, q], sem_w),
        ]

    def compute(q, s):
        (u_b, pos_v, neg_v, nbr_v, hu_b, nbr_b, plog_b, nlog_b, _, _) = sets[s]
        for r in range(CH):
            u = [u_b[r, pl.ds(c * NL, NL)] for c in range(NC)]
            goff = r * L

            def l_body(l, acc, r=r, u=u):
                a0, a1, a2, a3, plv, nlv = acc
                g = r * L + l
                pv = [pos_v[g, pl.ds(c * NL, NL)] for c in range(NC)]
                nv = [neg_v[g, pl.ds(c * NL, NL)] for c in range(NC)]
                bv = [nbr_v[g, pl.ds(c * NL, NL)] for c in range(NC)]
                ps = jnp.sum(u[0] * pv[0] + u[1] * pv[1]
                             + u[2] * pv[2] + u[3] * pv[3])
                ns = jnp.sum(u[0] * nv[0] + u[1] * nv[1]
                             + u[2] * nv[2] + u[3] * nv[3])
                eq = iota == (l % NL)
                plv = jnp.where(eq, ps, plv)
                nlv = jnp.where(eq, ns, nlv)
                slot = r * LP + (l // NL) * NL
                plog_b[pl.ds(slot, NL)] = plv
                nlog_b[pl.ds(slot, NL)] = nlv
                for c in range(NC):
                    hu_b[g, pl.ds(c * NL, NL)] = u[c]
                a0 = a0 + bv[0]
                a1 = a1 + bv[1]
                a2 = a2 + bv[2]
                a3 = a3 + bv[3]
                return (a0, a1, a2, a3, plv, nlv)

            z = jnp.zeros((NL,), _f32)
            a0, a1, a2, a3, _, _ = lax.fori_loop(
                0, L, l_body, (z, z, z, z, z, z))

            zc = jnp.zeros((NL,), jnp.int32)
            for k in range(3):
                bvix = bidx_v[q, pl.ds(goff + k * NL, NL)]
                zc = zc + jnp.where(bvix == 0, jnp.int32(1), jnp.int32(0))
            tail = bidx_v[q, pl.ds(goff + 34, NL)]
            tmask = (tail == 0) & (iota >= NL - 2)
            zc = zc + jnp.where(tmask, jnp.int32(1), jnp.int32(0))
            nzero = jnp.sum(zc)
            nzf = nzero.astype(_f32)
            cf = _f32(L) - nzf
            nonempty = nzero < L
            a = [a0, a1, a2, a3]
            m = [jnp.where(nonempty, (a[c] - nzf * u0[c]) / cf,
                           jnp.zeros((NL,), _f32) / cf)
                 for c in range(NC)]

            def fill_body(l, _, r=r, m=m):
                g = r * L + l
                for c in range(NC):
                    nbr_b[g, pl.ds(c * NL, NL)] = m[c]
                return 0

            lax.fori_loop(0, L, fill_body, 0)

    def chunk(q, s, wait_w, issue_g):
        for d in g_descs(q, s):
            d.wait()
        if wait_w:
            for d in w_descs(q - 2, (s + 1) % NSET):
                d.wait()
        if issue_g:
            for d in g_descs(q + 1, (s + 1) % NSET):
                d.start()
        compute(q, s)
        for d in w_descs(q, s):
            d.start()

    for d in g_descs(0, 0):
        d.start()
    chunk(0, 0, wait_w=False, issue_g=True)
    chunk(1, 1, wait_w=False, issue_g=True)
    chunk(2, 2, wait_w=True, issue_g=True)

    def ring_body(i, carry):
        q0 = 3 * i
        chunk(q0, 0, wait_w=True, issue_g=True)
        chunk(q0 + 1, 1, wait_w=True, issue_g=True)
        chunk(q0 + 2, 2, wait_w=True, issue_g=True)
        return carry

    lax.fori_loop(1, NSC // 3, ring_body, 0)

    chunk(NSC - 1, 0, wait_w=True, issue_g=False)
    for d in w_descs(NSC - 2, 2):
        d.wait()
    for d in w_descs(NSC - 1, 0):
        d.wait()


def kernel(uid, seq, pos, neg, nbr, nbr_iid, user_embs, item_embs):
    del seq, nbr_iid
    uid_r = uid.astype(jnp.int32).reshape(NW, NSC, CH)
    pos_r = pos.astype(jnp.int32).reshape(NW, NSC, SR)
    neg_r = neg.astype(jnp.int32).reshape(NW, NSC, SR)
    nbr_r = nbr.astype(jnp.int32).reshape(NW, NSC, SR)
    hu, pos_hi, neg_hi, nbr_emb, plog, nlog = _social_mf_sc(
        uid_r, pos_r, neg_r, nbr_r, user_embs, item_embs)
    return (
        plog.reshape(B, LP)[:, :L],
        nlog.reshape(B, LP)[:, :L],
        hu.reshape(B, L, D),
        pos_hi.reshape(B, L, D),
        neg_hi.reshape(B, L, D),
        nbr_emb.reshape(B, L, D),
    )
